# baseline (speedup 1.0000x reference)
import functools

import jax
import jax.numpy as jnp
from jax import lax
from jax.experimental import pallas as pl
from jax.experimental.pallas import tpu as pltpu
from jax.experimental.pallas import tpu_sc as plsc

NC = 2
NS = 16
NW = NC * NS
L = 16
C = 64
NBUF = 8
D = 128
DW = D // 2


def _bits_to_f32(w):
    return lax.bitcast_convert_type(w, jnp.float32)


def _shuffle(v, idx):
    dnums = lax.GatherDimensionNumbers(
        offset_dims=(), collapsed_slice_dims=(0,), start_index_map=(0,))
    return lax.gather(v, idx[:, None], dnums, slice_sizes=(1,),
                      mode=lax.GatherScatterMode.PROMISE_IN_BOUNDS)


def _build(n_edges_pad: int):
    bpw = n_edges_pad // NW
    n_chunks = bpw // C

    mesh = plsc.VectorSubcoreMesh(core_axis_name="c", subcore_axis_name="s")

    @functools.partial(
        pl.kernel,
        out_type=jax.ShapeDtypeStruct((n_edges_pad,), jnp.float32),
        mesh=mesh,
        compiler_params=pltpu.CompilerParams(use_tc_tiling_on_sc=False),
        scratch_types=[
            pltpu.VMEM((bpw,), jnp.int32),
            pltpu.VMEM((bpw,), jnp.int32),
            pltpu.VMEM((bpw,), jnp.float32),
            pltpu.VMEM((NBUF, C, DW), jnp.int32),
            pltpu.VMEM((NBUF, C, DW), jnp.int32),
        ] + [pltpu.SemaphoreType.DMA] * NBUF,
    )
    def launch(xa, xp, ia, ip, out, ia_v, ip_v, out_v, ra_v, rp_v, *sems):
        wid = lax.axis_index("s") * NC + lax.axis_index("c")
        base = wid * bpw
        pltpu.sync_copy(ia.at[pl.ds(base, bpw)], ia_v)
        pltpu.sync_copy(ip.at[pl.ds(base, bpw)], ip_v)

        row_ids = lax.iota(jnp.int32, L)

        def descs(ch, b):
            off = ch * C
            return (
                pltpu.make_async_copy(
                    xa.at[ia_v.at[pl.ds(off, C)]], ra_v.at[b], sems[b]),
                pltpu.make_async_copy(
                    xp.at[ip_v.at[pl.ds(off, C)]], rp_v.at[b], sems[b]),
            )

        def issue(ch, b):
            for d in descs(ch, b):
                d.start()

        def wait(ch, b):
            for d in descs(ch, b):
                d.wait()

        def compute(ch, b):
            ra = ra_v.at[b]
            rp = rp_v.at[b]
            off = ch * C

            def group_body(g, _):
                gbase = g * L
                s = jnp.zeros((L,), jnp.float32)
                for e in range(L):
                    r = gbase + e
                    acc = jnp.zeros((L,), jnp.float32)
                    for k in range(DW // L):
                        aw = ra[r, pl.ds(k * L, L)]
                        pw = rp[r, pl.ds(k * L, L)]
                        a_lo = _bits_to_f32(aw << 16)
                        a_hi = _bits_to_f32(aw & jnp.int32(-65536))
                        p_lo = _bits_to_f32(pw << 16)
                        p_hi = _bits_to_f32(pw & jnp.int32(-65536))
                        acc = acc + a_lo * p_lo + a_hi * p_hi
                    for sh in (8, 4, 2, 1):
                        acc = acc + _shuffle(acc, row_ids ^ sh)
                    s = jnp.where(row_ids == e, acc, s)
                out_v[pl.ds(off + gbase, L)] = s
                return 0

            lax.fori_loop(0, C // L, group_body, 0)

        for b in range(NBUF - 1):
            issue(b, b)

        def ring_body(i, _):
            for b in range(NBUF):
                ch = NBUF * i + b
                wait(ch, b)

                @pl.when(ch + NBUF - 1 < n_chunks)
                def _():
                    issue(ch + NBUF - 1, (b + NBUF - 1) % NBUF)

                compute(ch, b)
            return 0

        lax.fori_loop(0, n_chunks // NBUF, ring_body, 0)
        pltpu.sync_copy(out_v, out.at[pl.ds(base, bpw)])

    return launch


def kernel(x_author, x_paper, edge_index, supervision_edge_index):
    n = supervision_edge_index.shape[1]
    chunk_all = NW * C * NBUF
    n_pad = ((n + chunk_all - 1) // chunk_all) * chunk_all
    ids = supervision_edge_index.astype(jnp.int32)
    ia = jnp.pad(ids[0], (0, n_pad - n))
    ip = jnp.pad(ids[1], (0, n_pad - n))
    v, d = x_author.shape
    xa_packed = lax.bitcast_convert_type(
        x_author.astype(jnp.bfloat16).reshape(v, d // 2, 2), jnp.int32)
    xp_packed = lax.bitcast_convert_type(
        x_paper.astype(jnp.bfloat16).reshape(v, d // 2, 2), jnp.int32)
    scores = _build(n_pad)(xa_packed, xp_packed, ia, ip)
    return scores[:n]


# speedup vs baseline: 2.1187x; 2.1187x over previous
import functools

import jax
import jax.numpy as jnp
from jax import lax
from jax.experimental import pallas as pl
from jax.experimental.pallas import tpu as pltpu
from jax.experimental.pallas import tpu_sc as plsc

NC = 2
NS = 16
NW = NC * NS
L = 16
C = 64
NBUF = 8
D = 128
DW = D // 2


def _bits_to_f32(w):
    return lax.bitcast_convert_type(w, jnp.float32)


def _shuffle(v, idx):
    dnums = lax.GatherDimensionNumbers(
        offset_dims=(), collapsed_slice_dims=(0,), start_index_map=(0,))
    return lax.gather(v, idx[:, None], dnums, slice_sizes=(1,),
                      mode=lax.GatherScatterMode.PROMISE_IN_BOUNDS)


def _build(n_edges_pad: int):
    bpw = n_edges_pad // NW
    n_chunks = bpw // C

    mesh = plsc.VectorSubcoreMesh(core_axis_name="c", subcore_axis_name="s")

    @functools.partial(
        pl.kernel,
        out_type=jax.ShapeDtypeStruct((n_edges_pad,), jnp.float32),
        mesh=mesh,
        compiler_params=pltpu.CompilerParams(use_tc_tiling_on_sc=False),
        scratch_types=[
            pltpu.VMEM((bpw,), jnp.int32),
            pltpu.VMEM((bpw,), jnp.int32),
            pltpu.VMEM((bpw,), jnp.float32),
            pltpu.VMEM((NBUF, C, DW), jnp.int32),
            pltpu.VMEM((NBUF, C, DW), jnp.int32),
        ] + [pltpu.SemaphoreType.DMA] * NBUF,
    )
    def launch(xa, xp, ia, ip, out, ia_v, ip_v, out_v, ra_v, rp_v, *sems):
        wid = lax.axis_index("s") * NC + lax.axis_index("c")
        base = wid * bpw
        pltpu.sync_copy(ia.at[pl.ds(base, bpw)], ia_v)
        pltpu.sync_copy(ip.at[pl.ds(base, bpw)], ip_v)

        row_ids = lax.iota(jnp.int32, L)

        def descs(ch, b):
            off = ch * C
            return (
                pltpu.make_async_copy(
                    xa.at[ia_v.at[pl.ds(off, C)]], ra_v.at[b], sems[b]),
                pltpu.make_async_copy(
                    xp.at[ip_v.at[pl.ds(off, C)]], rp_v.at[b], sems[b]),
            )

        def issue(ch, b):
            for d in descs(ch, b):
                d.start()

        def wait(ch, b):
            for d in descs(ch, b):
                d.wait()

        def compute(ch, b):
            ra = ra_v.at[b]
            rp = rp_v.at[b]
            off = ch * C

            def group_body(g, _):
                gbase = g * L
                s = jnp.zeros((L,), jnp.float32)
                for e in range(L):
                    r = gbase + e
                    acc = jnp.zeros((L,), jnp.float32)
                    for k in range(DW // L):
                        aw = ra[r, pl.ds(k * L, L)]
                        pw = rp[r, pl.ds(k * L, L)]
                        a_lo = _bits_to_f32(aw << 16)
                        a_hi = _bits_to_f32(aw & jnp.int32(-65536))
                        p_lo = _bits_to_f32(pw << 16)
                        p_hi = _bits_to_f32(pw & jnp.int32(-65536))
                        acc = acc + a_lo * p_lo + a_hi * p_hi
                    for sh in (8, 4, 2, 1):
                        acc = acc + _shuffle(acc, row_ids ^ sh)
                    s = jnp.where(row_ids == e, acc, s)
                out_v[pl.ds(off + gbase, L)] = s
                return 0

            lax.fori_loop(0, C // L, group_body, 0)

        for b in range(NBUF - 1):
            issue(b, b)

        def ring_body(i, _):
            for b in range(NBUF):
                ch = NBUF * i + b
                wait(ch, b)

                @pl.when(ch + NBUF - 1 < n_chunks)
                def _():
                    issue(ch + NBUF - 1, (b + NBUF - 1) % NBUF)

                compute(ch, b)
            return 0

        lax.fori_loop(0, n_chunks // NBUF, ring_body, 0)
        pltpu.sync_copy(out_v, out.at[pl.ds(base, bpw)])

    return launch


def _pack_body(x_ref, o_ref):
    xb = lax.bitcast_convert_type(x_ref[...], jnp.int32)
    r = (xb + jnp.int32(0x7FFF) + ((xb >> 16) & 1)) >> 16
    o_ref[...] = (r[:, :DW] & jnp.int32(0xFFFF)) | (r[:, DW:] << 16)


def _pack_table(x):
    v, d = x.shape
    rows = 1000
    return pl.pallas_call(
        _pack_body,
        grid=(v // rows,),
        in_specs=[pl.BlockSpec((rows, d), lambda i: (i, 0))],
        out_specs=pl.BlockSpec((rows, d // 2), lambda i: (i, 0)),
        out_shape=jax.ShapeDtypeStruct((v, d // 2), jnp.int32),
    )(x)


def kernel(x_author, x_paper, edge_index, supervision_edge_index):
    n = supervision_edge_index.shape[1]
    chunk_all = NW * C * NBUF
    n_pad = ((n + chunk_all - 1) // chunk_all) * chunk_all
    ids = supervision_edge_index.astype(jnp.int32)
    ia = jnp.pad(ids[0], (0, n_pad - n))
    ip = jnp.pad(ids[1], (0, n_pad - n))
    scores = _build(n_pad)(_pack_table(x_author), _pack_table(x_paper), ia, ip)
    return scores[:n]


# speedup vs baseline: 2.2749x; 1.0737x over previous
import functools

import jax
import jax.numpy as jnp
from jax import lax
from jax.experimental import pallas as pl
from jax.experimental.pallas import tpu as pltpu
from jax.experimental.pallas import tpu_sc as plsc

NC = 2
NS = 16
NW = NC * NS
L = 16
C = 64
NBUF = 8
D = 128
DW = D // 2


def _bits_to_f32(w):
    return lax.bitcast_convert_type(w, jnp.float32)


def _shuffle(v, idx):
    dnums = lax.GatherDimensionNumbers(
        offset_dims=(), collapsed_slice_dims=(0,), start_index_map=(0,))
    return lax.gather(v, idx[:, None], dnums, slice_sizes=(1,),
                      mode=lax.GatherScatterMode.PROMISE_IN_BOUNDS)


def _build(n_edges: int):
    bpw = -(-n_edges // NW)
    bpw = -(-bpw // (C * NBUF)) * (C * NBUF)
    n_chunks = bpw // C
    assert bpw % 8 == 0 and (n_edges - bpw) % 8 == 0

    mesh = plsc.VectorSubcoreMesh(core_axis_name="c", subcore_axis_name="s")

    @functools.partial(
        pl.kernel,
        out_type=jax.ShapeDtypeStruct((n_edges,), jnp.float32),
        mesh=mesh,
        compiler_params=pltpu.CompilerParams(use_tc_tiling_on_sc=False),
        scratch_types=[
            pltpu.VMEM((bpw,), jnp.int32),
            pltpu.VMEM((bpw,), jnp.int32),
            pltpu.VMEM((bpw,), jnp.float32),
            pltpu.VMEM((NBUF, C, DW), jnp.int32),
            pltpu.VMEM((NBUF, C, DW), jnp.int32),
        ] + [pltpu.SemaphoreType.DMA] * NBUF,
    )
    def launch(xa, xp, sup, out, ia_v, ip_v, out_v, ra_v, rp_v, *sems):
        wid = lax.axis_index("s") * NC + lax.axis_index("c")
        base = pl.multiple_of(jnp.minimum(wid * bpw, n_edges - bpw), 8)
        pltpu.sync_copy(sup.at[0, pl.ds(base, bpw)], ia_v)
        pltpu.sync_copy(sup.at[1, pl.ds(base, bpw)], ip_v)

        row_ids = lax.iota(jnp.int32, L)

        def descs(ch, b):
            off = ch * C
            return (
                pltpu.make_async_copy(
                    xa.at[ia_v.at[pl.ds(off, C)]], ra_v.at[b], sems[b]),
                pltpu.make_async_copy(
                    xp.at[ip_v.at[pl.ds(off, C)]], rp_v.at[b], sems[b]),
            )

        def issue(ch, b):
            for d in descs(ch, b):
                d.start()

        def wait(ch, b):
            for d in descs(ch, b):
                d.wait()

        def compute(ch, b):
            ra = ra_v.at[b]
            rp = rp_v.at[b]
            off = ch * C

            def group_body(g, _):
                gbase = g * L
                s = jnp.zeros((L,), jnp.float32)
                for e in range(L):
                    r = gbase + e
                    acc = jnp.zeros((L,), jnp.float32)
                    for k in range(DW // L):
                        aw = ra[r, pl.ds(k * L, L)]
                        pw = rp[r, pl.ds(k * L, L)]
                        a_lo = _bits_to_f32(aw << 16)
                        a_hi = _bits_to_f32(aw & jnp.int32(-65536))
                        p_lo = _bits_to_f32(pw << 16)
                        p_hi = _bits_to_f32(pw & jnp.int32(-65536))
                        acc = acc + a_lo * p_lo + a_hi * p_hi
                    for sh in (8, 4, 2, 1):
                        acc = acc + _shuffle(acc, row_ids ^ sh)
                    s = jnp.where(row_ids == e, acc, s)
                out_v[pl.ds(off + gbase, L)] = s
                return 0

            lax.fori_loop(0, C // L, group_body, 0)

        for b in range(NBUF - 1):
            issue(b, b)

        def ring_body(i, _):
            for b in range(NBUF):
                ch = NBUF * i + b
                wait(ch, b)

                @pl.when(ch + NBUF - 1 < n_chunks)
                def _():
                    issue(ch + NBUF - 1, (b + NBUF - 1) % NBUF)

                compute(ch, b)
            return 0

        lax.fori_loop(0, n_chunks // NBUF, ring_body, 0)
        pltpu.sync_copy(out_v, out.at[pl.ds(base, bpw)])

    return launch


def _pack_body(x_ref, o_ref):
    xb = lax.bitcast_convert_type(x_ref[...], jnp.int32)
    r = (xb + jnp.int32(0x7FFF) + ((xb >> 16) & 1)) >> 16
    o_ref[...] = (r[:, :DW] & jnp.int32(0xFFFF)) | (r[:, DW:] << 16)


def _pack_table(x):
    v, d = x.shape
    rows = 1000
    return pl.pallas_call(
        _pack_body,
        grid=(v // rows,),
        in_specs=[pl.BlockSpec((rows, d), lambda i: (i, 0))],
        out_specs=pl.BlockSpec((rows, d // 2), lambda i: (i, 0)),
        out_shape=jax.ShapeDtypeStruct((v, d // 2), jnp.int32),
    )(x)


def kernel(x_author, x_paper, edge_index, supervision_edge_index):
    n = supervision_edge_index.shape[1]
    return _build(n)(_pack_table(x_author), _pack_table(x_paper),
                     supervision_edge_index.astype(jnp.int32))


# speedup vs baseline: 2.6873x; 1.1813x over previous
import functools

import jax
import jax.numpy as jnp
from jax import lax
from jax.experimental import pallas as pl
from jax.experimental.pallas import tpu as pltpu
from jax.experimental.pallas import tpu_sc as plsc

NC = 2
NS = 16
NW = NC * NS
L = 16
C = 64
NBUF = 8
D = 128
DW = D // 2


def _bits_to_f32(w):
    return lax.bitcast_convert_type(w, jnp.float32)


def _shuffle(v, idx):
    dnums = lax.GatherDimensionNumbers(
        offset_dims=(), collapsed_slice_dims=(0,), start_index_map=(0,))
    return lax.gather(v, idx[:, None], dnums, slice_sizes=(1,),
                      mode=lax.GatherScatterMode.PROMISE_IN_BOUNDS)


def _build(n_edges: int):
    bpw = -(-n_edges // NW)
    bpw = -(-bpw // (C * NBUF)) * (C * NBUF)
    n_chunks = bpw // C
    assert bpw % 8 == 0 and (n_edges - bpw) % 8 == 0

    mesh = plsc.VectorSubcoreMesh(core_axis_name="c", subcore_axis_name="s")

    @functools.partial(
        pl.kernel,
        out_type=jax.ShapeDtypeStruct((n_edges,), jnp.float32),
        mesh=mesh,
        compiler_params=pltpu.CompilerParams(use_tc_tiling_on_sc=False),
        scratch_types=[
            pltpu.VMEM((bpw,), jnp.int32),
            pltpu.VMEM((bpw,), jnp.int32),
            pltpu.VMEM((bpw,), jnp.float32),
            pltpu.VMEM((NBUF, C, DW), jnp.int32),
            pltpu.VMEM((NBUF, C, DW), jnp.int32),
        ] + [pltpu.SemaphoreType.DMA] * NBUF,
    )
    def launch(xa, xp, sup, out, ia_v, ip_v, out_v, ra_v, rp_v, *sems):
        wid = lax.axis_index("s") * NC + lax.axis_index("c")
        base = pl.multiple_of(jnp.minimum(wid * bpw, n_edges - bpw), 8)
        pltpu.sync_copy(sup.at[0, pl.ds(base, bpw)], ia_v)
        pltpu.sync_copy(sup.at[1, pl.ds(base, bpw)], ip_v)

        row_ids = lax.iota(jnp.int32, L)

        def descs(ch, b):
            off = ch * C
            return (
                pltpu.make_async_copy(
                    xa.at[ia_v.at[pl.ds(off, C)]], ra_v.at[b], sems[b]),
                pltpu.make_async_copy(
                    xp.at[ip_v.at[pl.ds(off, C)]], rp_v.at[b], sems[b]),
            )

        def issue(ch, b):
            for d in descs(ch, b):
                d.start()

        def wait(ch, b):
            for d in descs(ch, b):
                d.wait()

        def compute(ch, b):
            ra = ra_v.at[b]
            rp = rp_v.at[b]
            off = ch * C

            def group_body(g, _):
                gbase = g * L
                s = jnp.zeros((L,), jnp.float32)
                for e in range(L):
                    r = gbase + e
                    acc = jnp.zeros((L,), jnp.float32)
                    for k in range(DW // L):
                        aw = ra[r, pl.ds(k * L, L)]
                        pw = rp[r, pl.ds(k * L, L)]
                        a_lo = _bits_to_f32(aw << 16)
                        a_hi = _bits_to_f32(aw & jnp.int32(-65536))
                        p_lo = _bits_to_f32(pw << 16)
                        p_hi = _bits_to_f32(pw & jnp.int32(-65536))
                        acc = acc + a_lo * p_lo + a_hi * p_hi
                    for sh in (8, 4, 2, 1):
                        acc = acc + _shuffle(acc, row_ids ^ sh)
                    s = jnp.where(row_ids == e, acc, s)
                out_v[pl.ds(off + gbase, L)] = s
                return 0

            lax.fori_loop(0, C // L, group_body, 0)

        for b in range(NBUF - 1):
            issue(b, b)

        def ring_body(i, _):
            for b in range(NBUF):
                ch = NBUF * i + b
                wait(ch, b)

                @pl.when(ch + NBUF - 1 < n_chunks)
                def _():
                    issue(ch + NBUF - 1, (b + NBUF - 1) % NBUF)

                compute(ch, b)
            return 0

        lax.fori_loop(0, n_chunks // NBUF, ring_body, 0)
        pltpu.sync_copy(out_v, out.at[pl.ds(base, bpw)])

    return launch


def _pack_body(x_ref, o_ref):
    xb = lax.bitcast_convert_type(x_ref[...], jnp.int32)
    r = (xb + jnp.int32(0x7FFF) + ((xb >> 16) & 1)) >> 16
    o_ref[...] = (r[:, :DW] & jnp.int32(0xFFFF)) | (r[:, DW:] << 16)


def _pack_table(x):
    v, d = x.shape
    rows = 10000
    return pl.pallas_call(
        _pack_body,
        grid=(v // rows,),
        in_specs=[pl.BlockSpec((rows, d), lambda i: (i, 0))],
        out_specs=pl.BlockSpec((rows, d // 2), lambda i: (i, 0)),
        out_shape=jax.ShapeDtypeStruct((v, d // 2), jnp.int32),
    )(x)


def kernel(x_author, x_paper, edge_index, supervision_edge_index):
    n = supervision_edge_index.shape[1]
    return _build(n)(_pack_table(x_author), _pack_table(x_paper),
                     supervision_edge_index.astype(jnp.int32))


# speedup vs baseline: 3.4743x; 1.2928x over previous
import functools

import jax
import jax.numpy as jnp
from jax import lax
from jax.experimental import pallas as pl
from jax.experimental.pallas import tpu as pltpu
from jax.experimental.pallas import tpu_sc as plsc

NC = 2
NS = 16
NW = NC * NS
L = 16
C = 128
NBUF = 4
D = 128
DW = D // 2


def _bits_to_f32(w):
    return lax.bitcast_convert_type(w, jnp.float32)


def _shuffle(v, idx):
    dnums = lax.GatherDimensionNumbers(
        offset_dims=(), collapsed_slice_dims=(0,), start_index_map=(0,))
    return lax.gather(v, idx[:, None], dnums, slice_sizes=(1,),
                      mode=lax.GatherScatterMode.PROMISE_IN_BOUNDS)


def _build(n_edges: int):
    bpw = -(-n_edges // NW)
    bpw = -(-bpw // (C * NBUF)) * (C * NBUF)
    n_chunks = bpw // C
    assert bpw % 8 == 0 and (n_edges - bpw) % 8 == 0

    mesh = plsc.VectorSubcoreMesh(core_axis_name="c", subcore_axis_name="s")

    @functools.partial(
        pl.kernel,
        out_type=jax.ShapeDtypeStruct((n_edges,), jnp.float32),
        mesh=mesh,
        compiler_params=pltpu.CompilerParams(use_tc_tiling_on_sc=False),
        scratch_types=[
            pltpu.VMEM((bpw,), jnp.int32),
            pltpu.VMEM((bpw,), jnp.int32),
            pltpu.VMEM((bpw,), jnp.float32),
            pltpu.VMEM((NBUF, C, DW), jnp.int32),
            pltpu.VMEM((NBUF, C, DW), jnp.int32),
        ] + [pltpu.SemaphoreType.DMA] * NBUF,
    )
    def launch(xa, xp, sup, out, ia_v, ip_v, out_v, ra_v, rp_v, *sems):
        wid = lax.axis_index("s") * NC + lax.axis_index("c")
        base = pl.multiple_of(jnp.minimum(wid * bpw, n_edges - bpw), 8)
        pltpu.sync_copy(sup.at[0, pl.ds(base, bpw)], ia_v)
        pltpu.sync_copy(sup.at[1, pl.ds(base, bpw)], ip_v)

        row_ids = lax.iota(jnp.int32, L)

        def descs(ch, b):
            off = ch * C
            return (
                pltpu.make_async_copy(
                    xa.at[ia_v.at[pl.ds(off, C)]], ra_v.at[b], sems[b]),
                pltpu.make_async_copy(
                    xp.at[ip_v.at[pl.ds(off, C)]], rp_v.at[b], sems[b]),
            )

        def issue(ch, b):
            for d in descs(ch, b):
                d.start()

        def wait(ch, b):
            for d in descs(ch, b):
                d.wait()

        def compute(ch, b):
            ra = ra_v.at[b]
            rp = rp_v.at[b]
            off = ch * C

            def group_body(g, _):
                gbase = g * L
                s = jnp.zeros((L,), jnp.float32)
                for e in range(L):
                    r = gbase + e
                    acc = jnp.zeros((L,), jnp.float32)
                    for k in range(DW // L):
                        aw = ra[r, pl.ds(k * L, L)]
                        pw = rp[r, pl.ds(k * L, L)]
                        a_lo = _bits_to_f32(aw << 16)
                        a_hi = _bits_to_f32(aw & jnp.int32(-65536))
                        p_lo = _bits_to_f32(pw << 16)
                        p_hi = _bits_to_f32(pw & jnp.int32(-65536))
                        acc = acc + a_lo * p_lo + a_hi * p_hi
                    for sh in (8, 4, 2, 1):
                        acc = acc + _shuffle(acc, row_ids ^ sh)
                    s = jnp.where(row_ids == e, acc, s)
                out_v[pl.ds(off + gbase, L)] = s
                return 0

            lax.fori_loop(0, C // L, group_body, 0)

        for b in range(NBUF - 1):
            issue(b, b)

        def ring_body(i, _):
            for b in range(NBUF):
                ch = NBUF * i + b
                wait(ch, b)

                @pl.when(ch + NBUF - 1 < n_chunks)
                def _():
                    issue(ch + NBUF - 1, (b + NBUF - 1) % NBUF)

                compute(ch, b)
            return 0

        lax.fori_loop(0, n_chunks // NBUF, ring_body, 0)
        pltpu.sync_copy(out_v, out.at[pl.ds(base, bpw)])

    return launch


def _pack_body(x_ref, o_ref):
    xb = lax.bitcast_convert_type(x_ref[...], jnp.int32)
    r = (xb + jnp.int32(0x7FFF) + ((xb >> 16) & 1)) >> 16
    o_ref[...] = (r[:, :DW] & jnp.int32(0xFFFF)) | (r[:, DW:] << 16)


def _pack_table(x):
    v, d = x.shape
    rows = 10000
    return pl.pallas_call(
        _pack_body,
        grid=(v // rows,),
        in_specs=[pl.BlockSpec((rows, d), lambda i: (i, 0))],
        out_specs=pl.BlockSpec((rows, d // 2), lambda i: (i, 0)),
        out_shape=jax.ShapeDtypeStruct((v, d // 2), jnp.int32),
    )(x)


def kernel(x_author, x_paper, edge_index, supervision_edge_index):
    n = supervision_edge_index.shape[1]
    return _build(n)(_pack_table(x_author), _pack_table(x_paper),
                     supervision_edge_index.astype(jnp.int32))


# speedup vs baseline: 4.1632x; 1.1983x over previous
import functools

import jax
import jax.numpy as jnp
from jax import lax
from jax.experimental import pallas as pl
from jax.experimental.pallas import tpu as pltpu
from jax.experimental.pallas import tpu_sc as plsc

NC = 2
NS = 16
NW = NC * NS
L = 16
C = 256
NBUF = 2
D = 128
DW = D // 2


def _bits_to_f32(w):
    return lax.bitcast_convert_type(w, jnp.float32)


def _shuffle(v, idx):
    dnums = lax.GatherDimensionNumbers(
        offset_dims=(), collapsed_slice_dims=(0,), start_index_map=(0,))
    return lax.gather(v, idx[:, None], dnums, slice_sizes=(1,),
                      mode=lax.GatherScatterMode.PROMISE_IN_BOUNDS)


def _build(n_edges: int):
    bpw = -(-n_edges // NW)
    bpw = -(-bpw // (C * NBUF)) * (C * NBUF)
    n_chunks = bpw // C
    assert bpw % 8 == 0 and (n_edges - bpw) % 8 == 0

    mesh = plsc.VectorSubcoreMesh(core_axis_name="c", subcore_axis_name="s")

    @functools.partial(
        pl.kernel,
        out_type=jax.ShapeDtypeStruct((n_edges,), jnp.float32),
        mesh=mesh,
        compiler_params=pltpu.CompilerParams(use_tc_tiling_on_sc=False),
        scratch_types=[
            pltpu.VMEM((bpw,), jnp.int32),
            pltpu.VMEM((bpw,), jnp.int32),
            pltpu.VMEM((bpw,), jnp.float32),
            pltpu.VMEM((NBUF, C, DW), jnp.int32),
            pltpu.VMEM((NBUF, C, DW), jnp.int32),
        ] + [pltpu.SemaphoreType.DMA] * NBUF,
    )
    def launch(xa, xp, sup, out, ia_v, ip_v, out_v, ra_v, rp_v, *sems):
        wid = lax.axis_index("s") * NC + lax.axis_index("c")
        base = pl.multiple_of(jnp.minimum(wid * bpw, n_edges - bpw), 8)
        pltpu.sync_copy(sup.at[0, pl.ds(base, bpw)], ia_v)
        pltpu.sync_copy(sup.at[1, pl.ds(base, bpw)], ip_v)

        row_ids = lax.iota(jnp.int32, L)

        def descs(ch, b):
            off = ch * C
            return (
                pltpu.make_async_copy(
                    xa.at[ia_v.at[pl.ds(off, C)]], ra_v.at[b], sems[b]),
                pltpu.make_async_copy(
                    xp.at[ip_v.at[pl.ds(off, C)]], rp_v.at[b], sems[b]),
            )

        def issue(ch, b):
            for d in descs(ch, b):
                d.start()

        def wait(ch, b):
            for d in descs(ch, b):
                d.wait()

        def compute(ch, b):
            ra = ra_v.at[b]
            rp = rp_v.at[b]
            off = ch * C

            def group_body(g, _):
                gbase = g * L
                s = jnp.zeros((L,), jnp.float32)
                for e in range(L):
                    r = gbase + e
                    acc = jnp.zeros((L,), jnp.float32)
                    for k in range(DW // L):
                        aw = ra[r, pl.ds(k * L, L)]
                        pw = rp[r, pl.ds(k * L, L)]
                        a_lo = _bits_to_f32(aw << 16)
                        a_hi = _bits_to_f32(aw & jnp.int32(-65536))
                        p_lo = _bits_to_f32(pw << 16)
                        p_hi = _bits_to_f32(pw & jnp.int32(-65536))
                        acc = acc + a_lo * p_lo + a_hi * p_hi
                    for sh in (8, 4, 2, 1):
                        acc = acc + _shuffle(acc, row_ids ^ sh)
                    s = jnp.where(row_ids == e, acc, s)
                out_v[pl.ds(off + gbase, L)] = s
                return 0

            lax.fori_loop(0, C // L, group_body, 0)

        for b in range(NBUF - 1):
            issue(b, b)

        def ring_body(i, _):
            for b in range(NBUF):
                ch = NBUF * i + b
                wait(ch, b)

                @pl.when(ch + NBUF - 1 < n_chunks)
                def _():
                    issue(ch + NBUF - 1, (b + NBUF - 1) % NBUF)

                compute(ch, b)
            return 0

        lax.fori_loop(0, n_chunks // NBUF, ring_body, 0)
        pltpu.sync_copy(out_v, out.at[pl.ds(base, bpw)])

    return launch


def _pack_body(x_ref, o_ref):
    xb = lax.bitcast_convert_type(x_ref[...], jnp.int32)
    r = (xb + jnp.int32(0x7FFF) + ((xb >> 16) & 1)) >> 16
    o_ref[...] = (r[:, :DW] & jnp.int32(0xFFFF)) | (r[:, DW:] << 16)


def _pack_table(x):
    v, d = x.shape
    rows = 10000
    return pl.pallas_call(
        _pack_body,
        grid=(v // rows,),
        in_specs=[pl.BlockSpec((rows, d), lambda i: (i, 0))],
        out_specs=pl.BlockSpec((rows, d // 2), lambda i: (i, 0)),
        out_shape=jax.ShapeDtypeStruct((v, d // 2), jnp.int32),
    )(x)


def kernel(x_author, x_paper, edge_index, supervision_edge_index):
    n = supervision_edge_index.shape[1]
    return _build(n)(_pack_table(x_author), _pack_table(x_paper),
                     supervision_edge_index.astype(jnp.int32))


# speedup vs baseline: 4.2404x; 1.0185x over previous
import functools

import jax
import jax.numpy as jnp
from jax import lax
from jax.experimental import pallas as pl
from jax.experimental.pallas import tpu as pltpu
from jax.experimental.pallas import tpu_sc as plsc

NC = 2
NS = 16
NW = NC * NS
L = 16
C = 320
NBUF = 2
D = 128
DW = D // 2


def _bits_to_f32(w):
    return lax.bitcast_convert_type(w, jnp.float32)


def _shuffle(v, idx):
    dnums = lax.GatherDimensionNumbers(
        offset_dims=(), collapsed_slice_dims=(0,), start_index_map=(0,))
    return lax.gather(v, idx[:, None], dnums, slice_sizes=(1,),
                      mode=lax.GatherScatterMode.PROMISE_IN_BOUNDS)


def _build(n_edges: int):
    bpw = -(-n_edges // NW)
    bpw = -(-bpw // (C * NBUF)) * (C * NBUF)
    n_chunks = bpw // C
    assert bpw % 8 == 0 and (n_edges - bpw) % 8 == 0

    mesh = plsc.VectorSubcoreMesh(core_axis_name="c", subcore_axis_name="s")

    @functools.partial(
        pl.kernel,
        out_type=jax.ShapeDtypeStruct((n_edges,), jnp.float32),
        mesh=mesh,
        compiler_params=pltpu.CompilerParams(use_tc_tiling_on_sc=False),
        scratch_types=[
            pltpu.VMEM((bpw,), jnp.int32),
            pltpu.VMEM((bpw,), jnp.int32),
            pltpu.VMEM((bpw,), jnp.float32),
            pltpu.VMEM((NBUF, C, DW), jnp.int32),
            pltpu.VMEM((NBUF, C, DW), jnp.int32),
        ] + [pltpu.SemaphoreType.DMA] * NBUF,
    )
    def launch(xa, xp, sup, out, ia_v, ip_v, out_v, ra_v, rp_v, *sems):
        wid = lax.axis_index("s") * NC + lax.axis_index("c")
        base = pl.multiple_of(jnp.minimum(wid * bpw, n_edges - bpw), 8)
        pltpu.sync_copy(sup.at[0, pl.ds(base, bpw)], ia_v)
        pltpu.sync_copy(sup.at[1, pl.ds(base, bpw)], ip_v)

        row_ids = lax.iota(jnp.int32, L)

        def descs(ch, b):
            off = ch * C
            return (
                pltpu.make_async_copy(
                    xa.at[ia_v.at[pl.ds(off, C)]], ra_v.at[b], sems[b]),
                pltpu.make_async_copy(
                    xp.at[ip_v.at[pl.ds(off, C)]], rp_v.at[b], sems[b]),
            )

        def issue(ch, b):
            for d in descs(ch, b):
                d.start()

        def wait(ch, b):
            for d in descs(ch, b):
                d.wait()

        def compute(ch, b):
            ra = ra_v.at[b]
            rp = rp_v.at[b]
            off = ch * C

            def group_body(g, _):
                gbase = g * L
                s = jnp.zeros((L,), jnp.float32)
                for e in range(L):
                    r = gbase + e
                    acc = jnp.zeros((L,), jnp.float32)
                    for k in range(DW // L):
                        aw = ra[r, pl.ds(k * L, L)]
                        pw = rp[r, pl.ds(k * L, L)]
                        a_lo = _bits_to_f32(aw << 16)
                        a_hi = _bits_to_f32(aw & jnp.int32(-65536))
                        p_lo = _bits_to_f32(pw << 16)
                        p_hi = _bits_to_f32(pw & jnp.int32(-65536))
                        acc = acc + a_lo * p_lo + a_hi * p_hi
                    for sh in (8, 4, 2, 1):
                        acc = acc + _shuffle(acc, row_ids ^ sh)
                    s = jnp.where(row_ids == e, acc, s)
                out_v[pl.ds(off + gbase, L)] = s
                return 0

            lax.fori_loop(0, C // L, group_body, 0)

        for b in range(NBUF - 1):
            issue(b, b)

        def ring_body(i, _):
            for b in range(NBUF):
                ch = NBUF * i + b
                wait(ch, b)

                @pl.when(ch + NBUF - 1 < n_chunks)
                def _():
                    issue(ch + NBUF - 1, (b + NBUF - 1) % NBUF)

                compute(ch, b)
            return 0

        lax.fori_loop(0, n_chunks // NBUF, ring_body, 0)
        pltpu.sync_copy(out_v, out.at[pl.ds(base, bpw)])

    return launch


def _pack_body(x_ref, o_ref):
    xb = lax.bitcast_convert_type(x_ref[...], jnp.int32)
    r = (xb + jnp.int32(0x7FFF) + ((xb >> 16) & 1)) >> 16
    o_ref[...] = (r[:, :DW] & jnp.int32(0xFFFF)) | (r[:, DW:] << 16)


def _pack_table(x):
    v, d = x.shape
    rows = 10000
    return pl.pallas_call(
        _pack_body,
        grid=(v // rows,),
        in_specs=[pl.BlockSpec((rows, d), lambda i: (i, 0))],
        out_specs=pl.BlockSpec((rows, d // 2), lambda i: (i, 0)),
        out_shape=jax.ShapeDtypeStruct((v, d // 2), jnp.int32),
    )(x)


def kernel(x_author, x_paper, edge_index, supervision_edge_index):
    n = supervision_edge_index.shape[1]
    return _build(n)(_pack_table(x_author), _pack_table(x_paper),
                     supervision_edge_index.astype(jnp.int32))
